# baseline (speedup 1.0000x reference)
import jax
import jax.numpy as jnp
from jax import lax
from jax.experimental import pallas as pl
from jax.experimental.pallas import tpu as pltpu
from jax.experimental.pallas import tpu_sc as plsc

_G = 64
_T = 12
_BI = 8
_TC_ROWS = 40
_NBLK = _TC_ROWS // _BI
_SC_ROW0 = _TC_ROWS
_NW = 32
_SLABS_W = (_G - _SC_ROW0) * _G // _NW
_GRP = 4
_NGRP = _SLABS_W // _GRP


def _tc_reduce_kernel(od_ref, w_ref, out_ref, acc_b, acc_c):
    k = pl.program_id(0)

    @pl.when(k == 0)
    def _init():
        acc_b[...] = jnp.zeros_like(acc_b)
        acc_c[...] = jnp.zeros_like(acc_c)

    def _col(j, accs):
        ab, ac = accs
        for i in range(_BI):
            slab = od_ref[i, j]
            w = w_ref[k * _BI + i, j]
            ab = ab + w * slab
            ac = ac + slab
        return (ab, ac)

    ab, ac = lax.fori_loop(0, _G, _col, (acc_b[...], acc_c[...]))
    acc_b[...] = ab
    acc_c[...] = ac

    @pl.when(k == _NBLK - 1)
    def _finish():
        out_ref[0, :, :] = ab
        out_ref[1, :, :] = ac


def _sc_partial_body(od_hbm, mu_hbm, out_hbm, buf0, buf1, acc_b, acc_c,
                     mu_v, sem0, sem1):
    wid = lax.axis_index("s") * 2 + lax.axis_index("c")
    s0 = wid * _SLABS_W

    pltpu.sync_copy(mu_hbm.at[pl.ds(_SC_ROW0 * _G + s0, _SLABS_W)],
                    mu_v.at[pl.ds(0, _SLABS_W)])

    zeros16 = jnp.zeros((16,), jnp.float32)

    def _zero(r, carry):
        for c4 in range(4):
            acc_b[r, pl.ds(c4 * 16, 16)] = zeros16
            acc_c[r, pl.ds(c4 * 16, 16)] = zeros16
        return carry

    lax.fori_loop(0, _G, _zero, 0)

    bufs = (buf0, buf1)
    sems = (sem0, sem1)

    def _start(g):
        return pltpu.async_copy(
            od_hbm.at[pl.ds(_SC_ROW0 * _G + s0 + g * _GRP, _GRP)],
            bufs[g % 2], sems[g % 2])

    handles = {0: _start(0)}
    for g in range(_NGRP):
        if g + 1 < _NGRP:
            handles[g + 1] = _start(g + 1)
        handles[g].wait()
        buf = bufs[g % 2]

        ws = tuple(mu_v[pl.ds(g * _GRP + n, 16)][0] for n in range(_GRP))

        def _stripe(st, carry, buf=buf, ws=ws):
            r0 = st * 2
            avs = []
            for dr in range(2):
                for c4 in range(4):
                    avs.append(acc_b[r0 + dr, pl.ds(c4 * 16, 16)])
            for dr in range(2):
                for c4 in range(4):
                    avs.append(acc_c[r0 + dr, pl.ds(c4 * 16, 16)])
            for n in range(_GRP):
                w = ws[n]
                ods = []
                for dr in range(2):
                    for c4 in range(4):
                        ods.append(buf[n, r0 + dr, pl.ds(c4 * 16, 16)])
                avs = ([avs[p] + w * ods[p] for p in range(8)]
                       + [avs[8 + p] + ods[p] for p in range(8)])
            p = 0
            for dr in range(2):
                for c4 in range(4):
                    acc_b[r0 + dr, pl.ds(c4 * 16, 16)] = avs[p]
                    p += 1
            for dr in range(2):
                for c4 in range(4):
                    acc_c[r0 + dr, pl.ds(c4 * 16, 16)] = avs[p]
                    p += 1
            return carry

        lax.fori_loop(0, _G // 2, _stripe, 0)

    pltpu.sync_copy(acc_b, out_hbm.at[wid, 0])
    pltpu.sync_copy(acc_c, out_hbm.at[wid, 1])


_sc_partial = pl.kernel(
    _sc_partial_body,
    out_type=jax.ShapeDtypeStruct((_NW, 2, _G, _G), jnp.float32),
    mesh=plsc.VectorSubcoreMesh(core_axis_name="c", subcore_axis_name="s"),
    scratch_types=[
        pltpu.VMEM((_GRP, _G, _G), jnp.float32),
        pltpu.VMEM((_GRP, _G, _G), jnp.float32),
        pltpu.VMEM((_G, _G), jnp.float32),
        pltpu.VMEM((_G, _G), jnp.float32),
        pltpu.VMEM((_SLABS_W + 16,), jnp.float32),
        pltpu.SemaphoreType.DMA,
        pltpu.SemaphoreType.DMA,
    ],
    compiler_params=pltpu.CompilerParams(use_tc_tiling_on_sc=True),
)


def _combine_kernel(tcp_ref, scp_ref, mu0_ref, t_ref, ak_ref, bk_ref,
                    out_ref):
    base = tcp_ref[0] + jnp.sum(scp_ref[:, 0], axis=0)
    colsum = tcp_ref[1] + jnp.sum(scp_ref[:, 1], axis=0)
    t_norm = t_ref[...] * (2.0 * jnp.pi / 120.0)
    harm = (ak_ref[0] * jnp.sin(t_norm) + bk_ref[0] * jnp.cos(t_norm)
            + ak_ref[1] * jnp.sin(2.0 * t_norm)
            + bk_ref[1] * jnp.cos(2.0 * t_norm))
    harm3 = harm[:, 0].reshape(_T, 1, 1)
    mu0 = mu0_ref[...]
    out_ref[:, 0, :, :] = mu0[None, :, :] + harm3
    out_ref[:, 1, :, :] = base[None, :, :] + harm3 * colsum[None, :, :]


def kernel(t_array, mu0, a_k, b_k, od_matrix):
    od3 = od_matrix.reshape(_G * _G, _G, _G)
    mu_flat = mu0.reshape(_G * _G)
    t128 = jnp.broadcast_to(t_array[:, None], (_T, 128))

    sc_parts = _sc_partial(od3, mu_flat)

    tc_part = pl.pallas_call(
        _tc_reduce_kernel,
        grid=(_NBLK,),
        in_specs=[
            pl.BlockSpec((_BI, _G, _G, _G), lambda k: (k, 0, 0, 0)),
            pl.BlockSpec((_G, _G), lambda k: (0, 0),
                         memory_space=pltpu.SMEM),
        ],
        out_specs=pl.BlockSpec((2, _G, _G), lambda k: (0, 0, 0)),
        out_shape=jax.ShapeDtypeStruct((2, _G, _G), jnp.float32),
        scratch_shapes=[
            pltpu.VMEM((_G, _G), jnp.float32),
            pltpu.VMEM((_G, _G), jnp.float32),
        ],
        compiler_params=pltpu.CompilerParams(
            dimension_semantics=("arbitrary",)),
    )(od_matrix, mu0)

    out = pl.pallas_call(
        _combine_kernel,
        in_specs=[
            pl.BlockSpec((2, _G, _G), lambda: (0, 0, 0)),
            pl.BlockSpec((_NW, 2, _G, _G), lambda: (0, 0, 0, 0)),
            pl.BlockSpec((_G, _G), lambda: (0, 0)),
            pl.BlockSpec((_T, 128), lambda: (0, 0)),
            pl.BlockSpec(memory_space=pltpu.SMEM),
            pl.BlockSpec(memory_space=pltpu.SMEM),
        ],
        out_specs=pl.BlockSpec((_T, 2, _G, _G), lambda: (0, 0, 0, 0)),
        out_shape=jax.ShapeDtypeStruct((_T, 2, _G, _G), jnp.float32),
    )(tc_part, sc_parts, mu0, t128, a_k, b_k)

    return out
